# baseline (speedup 1.0000x reference)
import jax
import jax.numpy as jnp
from jax import lax
from jax.experimental import pallas as pl
from jax.experimental.pallas import tpu as pltpu
from jax.experimental.pallas import tpu_sc as plsc

TOPK = 8
NUM_WORKERS = 32


def _stats_kernel(xt_ref, pk_ref, sim_ref, idx_ref, xn_ref, pn_ref, rs_ref,
                  means_ref):
    i = pl.program_id(0)
    rows = xt_ref.shape[1]
    n = xt_ref.shape[0]
    p = pk_ref.shape[0]
    b = means_ref.shape[0]

    x = xt_ref[...]
    means_ref[pl.ds(i * rows, rows), :] = jnp.sum(x, axis=0) / jnp.float32(n)

    @pl.when(i == pl.num_programs(0) - 1)
    def _tail():
        pk = pk_ref[...]
        pss = jnp.sum(pk * pk, axis=1, keepdims=True)
        pn = pk * lax.rsqrt(jnp.maximum(pss, jnp.float32(1e-12)))
        pn_ref[...] = pn

        mm = means_ref[...]
        mss = jnp.sum(mm * mm, axis=1, keepdims=True)
        xn = mm * lax.rsqrt(jnp.maximum(mss, jnp.float32(1e-12)))
        xn_ref[...] = xn

        sim = lax.dot_general(
            xn, pn, (((1,), (1,)), ((), ())),
            precision=lax.Precision.DEFAULT,
            preferred_element_type=jnp.float32)
        sim_ref[...] = sim

        iota = lax.broadcasted_iota(jnp.int32, (b, p), 1)
        kiota = lax.broadcasted_iota(jnp.int32, (b, TOPK), 1)
        vals = sim
        idx_acc = jnp.zeros((b, TOPK), jnp.int32)
        ssum = jnp.float32(0.0)
        for k in range(TOPK):
            m = jnp.max(vals, axis=1, keepdims=True)
            im = jnp.min(jnp.where(vals == m, iota, jnp.int32(p)),
                         axis=1, keepdims=True)
            idx_acc = jnp.where(kiota == k, im, idx_acc)
            ssum = ssum + jnp.sum(m)
            vals = jnp.where(iota == im, -jnp.inf, vals)
        idx_ref[...] = idx_acc
        rs_ref[...] = jnp.full((1, 1), ssum / jnp.float32(b), jnp.float32)


def _make_assemble(b, n, c, p, length):
    grows = TOPK * length
    out_rows = grows + n
    qsz = b // 8
    nq = b // qsz
    lanes = 16
    gpw = grows // NUM_WORKERS
    cpw = -(-n // NUM_WORKERS)
    bpw = b // NUM_WORKERS

    mesh = plsc.VectorSubcoreMesh(core_axis_name="c", subcore_axis_name="s",
                                  num_cores=2, num_subcores=16)

    def body(xt_hbm, prompt_hbm, pn_hbm, idx_hbm, kcm_hbm, bqm_hbm, out_hbm,
             bkn_hbm):
        wid = lax.axis_index("s") * 2 + lax.axis_index("c")

        def run(h0, h1, idxv2, idxg, idxb, kcv, bqv, gsem, wsem):
            bufs = (h0, h1)
            pltpu.sync_copy(kcm_hbm, kcv)
            pltpu.sync_copy(bqm_hbm, bqv)
            pltpu.sync_copy(idx_hbm, idxv2)

            for i in range(gpw):
                g = wid * gpw + i
                kk = g // jnp.int32(length)
                rr = g - kk * jnp.int32(length)
                kv = jnp.broadcast_to(kk, (lanes,))
                for ch in range(b // lanes):
                    bbv = lax.iota(jnp.int32, lanes) + jnp.int32(ch * lanes)
                    vals = plsc.load_gather(idxv2, [bbv, kv])
                    idxg[pl.ds(i * b + ch * lanes, lanes)] = (
                        rr * jnp.int32(p) + vals)
            for grp in range(bpw // 4):
                bb0 = wid * bpw + grp * 4
                for ch in range(2):
                    bv = jnp.broadcast_to(bb0 + 2 * ch, (lanes,)) + bqv[...]
                    vals = plsc.load_gather(idxv2, [bv, kcv[...]])
                    idxb[pl.ds(grp * 32 + ch * lanes, lanes)] = vals

            chunks = []
            for i in range(gpw):
                g = wid * gpw + i
                for q in range(nq):
                    chunks.append((
                        lambda i=i, q=q: prompt_hbm.at[
                            idxg.at[pl.ds(i * b + q * qsz, qsz)]],
                        lambda g=g, q=q: out_hbm.at[g, pl.ds(q * qsz, qsz), :],
                    ))
            for i in range(cpw):
                s = jnp.minimum(jnp.int32(NUM_WORKERS * i) + wid,
                                jnp.int32(n - 1))
                for q in range(nq):
                    chunks.append((
                        lambda s=s, q=q: xt_hbm.at[s, pl.ds(q * qsz, qsz), :],
                        lambda s=s, q=q: out_hbm.at[grows + s,
                                                    pl.ds(q * qsz, qsz), :],
                    ))
            for grp in range(bpw // 4):
                bb0 = wid * bpw + grp * 4
                chunks.append((
                    lambda grp=grp: pn_hbm.at[
                        idxb.at[pl.ds(grp * 32, 32)]],
                    lambda bb0=bb0: bkn_hbm.at[pl.ds(bb0 * TOPK, 32), :],
                ))

            pend = [None, None]
            for t, (src_fn, dst_fn) in enumerate(chunks):
                buf = bufs[t % 2]
                if pend[t % 2] is not None:
                    pend[t % 2].wait()
                pltpu.async_copy(src_fn(), buf, gsem).wait()
                pend[t % 2] = pltpu.async_copy(buf, dst_fn(), wsem)
            pend[0].wait()
            pend[1].wait()

        pl.run_scoped(run,
                      pltpu.VMEM((qsz, c), jnp.float32),
                      pltpu.VMEM((qsz, c), jnp.float32),
                      pltpu.VMEM((b, TOPK), jnp.int32),
                      pltpu.VMEM((gpw * b,), jnp.int32),
                      pltpu.VMEM((2 * 32,), jnp.int32),
                      pltpu.VMEM((lanes,), jnp.int32),
                      pltpu.VMEM((lanes,), jnp.int32),
                      pltpu.SemaphoreType.DMA,
                      pltpu.SemaphoreType.DMA)

    return pl.kernel(
        body,
        out_type=(
            jax.ShapeDtypeStruct((out_rows, b, c), jnp.float32),
            jax.ShapeDtypeStruct((b * TOPK, c), jnp.float32),
        ),
        mesh=mesh,
        compiler_params=pltpu.CompilerParams(needs_layout_passes=False),
    )


def kernel(x_embed, prompt_key, prompt):
    b, n, c = x_embed.shape
    p = prompt_key.shape[0]
    length = prompt.shape[1]
    chunk = b // 8

    xt = jnp.transpose(x_embed, (1, 0, 2))
    prompt_t = jnp.transpose(prompt, (1, 0, 2))
    prompt2d = prompt_t.reshape(length * p, c)

    in_specs = [
        pl.BlockSpec((n, chunk, c), lambda i: (0, i, 0)),
        pl.BlockSpec((p, c), lambda i: (0, 0)),
    ]
    out_shapes = (
        jax.ShapeDtypeStruct((b, p), jnp.float32),
        jax.ShapeDtypeStruct((b, TOPK), jnp.int32),
        jax.ShapeDtypeStruct((b, c), jnp.float32),
        jax.ShapeDtypeStruct((p, c), jnp.float32),
        jax.ShapeDtypeStruct((1, 1), jnp.float32),
    )
    out_specs = (
        pl.BlockSpec((b, p), lambda i: (0, 0)),
        pl.BlockSpec((b, TOPK), lambda i: (0, 0)),
        pl.BlockSpec((b, c), lambda i: (0, 0)),
        pl.BlockSpec((p, c), lambda i: (0, 0)),
        pl.BlockSpec((1, 1), lambda i: (0, 0)),
    )
    sim, idx, xn, pn, rs = pl.pallas_call(
        _stats_kernel,
        grid=(b // chunk,),
        in_specs=in_specs,
        out_specs=out_specs,
        out_shape=out_shapes,
        scratch_shapes=[pltpu.VMEM((b, c), jnp.float32)],
    )(xt, prompt_key)

    kcm = jnp.arange(16, dtype=jnp.int32) % jnp.int32(TOPK)
    bqm = jnp.arange(16, dtype=jnp.int32) // jnp.int32(TOPK)
    assemble = _make_assemble(b, n, c, p, length)
    outt, bkn2 = assemble(xt, prompt2d, pn, idx, kcm, bqm)
    prompted = jnp.transpose(outt, (1, 0, 2))
    bkn = bkn2.reshape(b, TOPK, c)

    return (prompted,
            sim,
            rs.reshape(()),
            idx,
            pn,
            xn,
            bkn)


# speedup vs baseline: 1.1378x; 1.1378x over previous
import jax
import jax.numpy as jnp
from jax import lax
from jax.experimental import pallas as pl
from jax.experimental.pallas import tpu as pltpu
from jax.experimental.pallas import tpu_sc as plsc

TOPK = 8
NUM_WORKERS = 32


def _stats_kernel(xt_ref, pk_ref, sim_ref, idx_ref, xn_ref, pn_ref, rs_ref,
                  means_ref):
    i = pl.program_id(0)
    rows = xt_ref.shape[1]
    n = xt_ref.shape[0]
    p = pk_ref.shape[0]
    b = means_ref.shape[0]

    x = xt_ref[...]
    means_ref[pl.ds(i * rows, rows), :] = jnp.sum(x, axis=0) / jnp.float32(n)

    @pl.when(i == pl.num_programs(0) - 1)
    def _tail():
        pk = pk_ref[...]
        pss = jnp.sum(pk * pk, axis=1, keepdims=True)
        pn = pk * lax.rsqrt(jnp.maximum(pss, jnp.float32(1e-12)))
        pn_ref[...] = pn

        mm = means_ref[...]
        mss = jnp.sum(mm * mm, axis=1, keepdims=True)
        xn = mm * lax.rsqrt(jnp.maximum(mss, jnp.float32(1e-12)))
        xn_ref[...] = xn

        sim = lax.dot_general(
            xn, pn, (((1,), (1,)), ((), ())),
            precision=lax.Precision.DEFAULT,
            preferred_element_type=jnp.float32)
        sim_ref[...] = sim

        iota = lax.broadcasted_iota(jnp.int32, (b, p), 1)
        kiota = lax.broadcasted_iota(jnp.int32, (b, TOPK), 1)
        vals = sim
        idx_acc = jnp.zeros((b, TOPK), jnp.int32)
        ssum = jnp.float32(0.0)
        for k in range(TOPK):
            m = jnp.max(vals, axis=1, keepdims=True)
            im = jnp.min(jnp.where(vals == m, iota, jnp.int32(p)),
                         axis=1, keepdims=True)
            idx_acc = jnp.where(kiota == k, im, idx_acc)
            ssum = ssum + jnp.sum(m)
            vals = jnp.where(iota == im, -jnp.inf, vals)
        idx_ref[...] = idx_acc
        rs_ref[...] = jnp.full((1, 1), ssum / jnp.float32(b), jnp.float32)


def _make_assemble(b, n, c, p, length):
    grows = TOPK * length
    out_rows = grows + n
    qsz = b // 8
    nq = b // qsz
    lanes = 16
    gpw = grows // NUM_WORKERS
    cpw = -(-n // NUM_WORKERS)
    bpw = b // NUM_WORKERS

    mesh = plsc.VectorSubcoreMesh(core_axis_name="c", subcore_axis_name="s",
                                  num_cores=2, num_subcores=16)

    def body(prompt_hbm, pn_hbm, idx_hbm, kcm_hbm, bqm_hbm, out_hbm,
             bkn_hbm):
        wid = lax.axis_index("s") * 2 + lax.axis_index("c")

        def run(h0, h1, idxv2, idxg, idxb, kcv, bqv, gsem, wsem):
            bufs = (h0, h1)
            pltpu.sync_copy(kcm_hbm, kcv)
            pltpu.sync_copy(bqm_hbm, bqv)
            pltpu.sync_copy(idx_hbm, idxv2)

            for i in range(gpw):
                g = wid * gpw + i
                kk = g // jnp.int32(length)
                rr = g - kk * jnp.int32(length)
                kv = jnp.broadcast_to(kk, (lanes,))
                for ch in range(b // lanes):
                    bbv = lax.iota(jnp.int32, lanes) + jnp.int32(ch * lanes)
                    vals = plsc.load_gather(idxv2, [bbv, kv])
                    idxg[pl.ds(i * b + ch * lanes, lanes)] = (
                        rr * jnp.int32(p) + vals)
            for grp in range(bpw // 4):
                bb0 = wid * bpw + grp * 4
                for ch in range(2):
                    bv = jnp.broadcast_to(bb0 + 2 * ch, (lanes,)) + bqv[...]
                    vals = plsc.load_gather(idxv2, [bv, kcv[...]])
                    idxb[pl.ds(grp * 32 + ch * lanes, lanes)] = vals

            chunks = []
            for i in range(gpw):
                g = wid * gpw + i
                for q in range(nq):
                    chunks.append((
                        lambda i=i, q=q: prompt_hbm.at[
                            idxg.at[pl.ds(i * b + q * qsz, qsz)]],
                        lambda g=g, q=q: out_hbm.at[g, pl.ds(q * qsz, qsz), :],
                    ))
            for grp in range(bpw // 4):
                bb0 = wid * bpw + grp * 4
                chunks.append((
                    lambda grp=grp: pn_hbm.at[
                        idxb.at[pl.ds(grp * 32, 32)]],
                    lambda bb0=bb0: bkn_hbm.at[pl.ds(bb0 * TOPK, 32), :],
                ))

            pend = [None, None]
            for t, (src_fn, dst_fn) in enumerate(chunks):
                buf = bufs[t % 2]
                if pend[t % 2] is not None:
                    pend[t % 2].wait()
                pltpu.async_copy(src_fn(), buf, gsem).wait()
                pend[t % 2] = pltpu.async_copy(buf, dst_fn(), wsem)
            pend[0].wait()
            pend[1].wait()

        pl.run_scoped(run,
                      pltpu.VMEM((qsz, c), jnp.float32),
                      pltpu.VMEM((qsz, c), jnp.float32),
                      pltpu.VMEM((b, TOPK), jnp.int32),
                      pltpu.VMEM((gpw * b,), jnp.int32),
                      pltpu.VMEM((2 * 32,), jnp.int32),
                      pltpu.VMEM((lanes,), jnp.int32),
                      pltpu.VMEM((lanes,), jnp.int32),
                      pltpu.SemaphoreType.DMA,
                      pltpu.SemaphoreType.DMA)

    return pl.kernel(
        body,
        out_type=(
            jax.ShapeDtypeStruct((out_rows, b, c), jnp.float32),
            jax.ShapeDtypeStruct((b * TOPK, c), jnp.float32),
        ),
        mesh=mesh,
        compiler_params=pltpu.CompilerParams(needs_layout_passes=False),
    )


def _patch_kernel(x_ref, outa_ref, out_ref):
    del outa_ref
    out_ref[...] = x_ref[...]


def kernel(x_embed, prompt_key, prompt):
    b, n, c = x_embed.shape
    p = prompt_key.shape[0]
    length = prompt.shape[1]
    chunk = b // 8

    xt = jnp.transpose(x_embed, (1, 0, 2))
    prompt_t = jnp.transpose(prompt, (1, 0, 2))
    prompt2d = prompt_t.reshape(length * p, c)

    in_specs = [
        pl.BlockSpec((n, chunk, c), lambda i: (0, i, 0)),
        pl.BlockSpec((p, c), lambda i: (0, 0)),
    ]
    out_shapes = (
        jax.ShapeDtypeStruct((b, p), jnp.float32),
        jax.ShapeDtypeStruct((b, TOPK), jnp.int32),
        jax.ShapeDtypeStruct((b, c), jnp.float32),
        jax.ShapeDtypeStruct((p, c), jnp.float32),
        jax.ShapeDtypeStruct((1, 1), jnp.float32),
    )
    out_specs = (
        pl.BlockSpec((b, p), lambda i: (0, 0)),
        pl.BlockSpec((b, TOPK), lambda i: (0, 0)),
        pl.BlockSpec((b, c), lambda i: (0, 0)),
        pl.BlockSpec((p, c), lambda i: (0, 0)),
        pl.BlockSpec((1, 1), lambda i: (0, 0)),
    )
    sim, idx, xn, pn, rs = pl.pallas_call(
        _stats_kernel,
        grid=(b // chunk,),
        in_specs=in_specs,
        out_specs=out_specs,
        out_shape=out_shapes,
        scratch_shapes=[pltpu.VMEM((b, c), jnp.float32)],
    )(xt, prompt_key)

    kcm = jnp.arange(16, dtype=jnp.int32) % jnp.int32(TOPK)
    bqm = jnp.arange(16, dtype=jnp.int32) // jnp.int32(TOPK)
    assemble = _make_assemble(b, n, c, p, length)
    outt, bkn2 = assemble(prompt2d, pn, idx, kcm, bqm)
    bkn = bkn2.reshape(b, TOPK, c)

    grows = TOPK * length
    out_rows = grows + n
    outt = pl.pallas_call(
        _patch_kernel,
        grid=(n // 4,),
        in_specs=[
            pl.BlockSpec((4, b, c), lambda i: (i, 0, 0)),
            pl.BlockSpec(memory_space=pltpu.HBM),
        ],
        out_specs=pl.BlockSpec((4, b, c), lambda i: (grows // 4 + i, 0, 0)),
        out_shape=jax.ShapeDtypeStruct((out_rows, b, c), jnp.float32),
        input_output_aliases={1: 0},
    )(xt, outt)
    prompted = jnp.transpose(outt, (1, 0, 2))

    return (prompted,
            sim,
            rs.reshape(()),
            idx,
            pn,
            xn,
            bkn)


# speedup vs baseline: 1.3559x; 1.1918x over previous
import functools

import jax
import jax.numpy as jnp
from jax import lax
from jax.experimental import pallas as pl
from jax.experimental.pallas import tpu as pltpu
from jax.experimental.pallas import tpu_sc as plsc

TOPK = 8
NUM_WORKERS = 32


def _stats_kernel(xt_ref, pk_ref, out_ref, sim_ref, idx_ref, xn_ref, pn_ref,
                  rs_ref, means_ref, *, n):
    i = pl.program_id(0)
    p = pk_ref.shape[0]
    b = means_ref.shape[0]

    x = xt_ref[...]
    out_ref[...] = x

    @pl.when(i == 0)
    def _init():
        means_ref[...] = jnp.zeros_like(means_ref)

    means_ref[...] += jnp.sum(x, axis=0)

    @pl.when(i == pl.num_programs(0) - 1)
    def _tail():
        pk = pk_ref[...]
        pss = jnp.sum(pk * pk, axis=1, keepdims=True)
        pn = pk * lax.rsqrt(jnp.maximum(pss, jnp.float32(1e-12)))
        pn_ref[...] = pn

        mm = means_ref[...] / jnp.float32(n)
        mss = jnp.sum(mm * mm, axis=1, keepdims=True)
        xn = mm * lax.rsqrt(jnp.maximum(mss, jnp.float32(1e-12)))
        xn_ref[...] = xn

        sim = lax.dot_general(
            xn, pn, (((1,), (1,)), ((), ())),
            precision=lax.Precision.DEFAULT,
            preferred_element_type=jnp.float32)
        sim_ref[...] = sim

        iota = lax.broadcasted_iota(jnp.int32, (b, p), 1)
        kiota = lax.broadcasted_iota(jnp.int32, (b, TOPK), 1)
        vals = sim
        idx_acc = jnp.zeros((b, TOPK), jnp.int32)
        ssum = jnp.float32(0.0)
        for k in range(TOPK):
            m = jnp.max(vals, axis=1, keepdims=True)
            im = jnp.min(jnp.where(vals == m, iota, jnp.int32(p)),
                         axis=1, keepdims=True)
            idx_acc = jnp.where(kiota == k, im, idx_acc)
            ssum = ssum + jnp.sum(m)
            vals = jnp.where(iota == im, -jnp.inf, vals)
        idx_ref[...] = idx_acc
        rs_ref[...] = jnp.full((1, 1), ssum / jnp.float32(b), jnp.float32)


def _make_assemble(b, n, c, p, length):
    grows = TOPK * length
    out_rows = grows + n
    qsz = b // 8
    nq = b // qsz
    lanes = 16
    gpw = grows // NUM_WORKERS
    cpw = -(-n // NUM_WORKERS)
    bpw = b // NUM_WORKERS

    mesh = plsc.VectorSubcoreMesh(core_axis_name="c", subcore_axis_name="s",
                                  num_cores=2, num_subcores=16)

    def body(prompt_hbm, pn_hbm, idx_hbm, kcm_hbm, bqm_hbm, out_hbm,
             bkn_hbm):
        wid = lax.axis_index("s") * 2 + lax.axis_index("c")

        def run(h0, h1, idxv2, idxg, idxb, kcv, bqv, gsem, wsem):
            bufs = (h0, h1)
            pltpu.sync_copy(kcm_hbm, kcv)
            pltpu.sync_copy(bqm_hbm, bqv)
            pltpu.sync_copy(idx_hbm, idxv2)

            for i in range(gpw):
                g = wid * gpw + i
                kk = g // jnp.int32(length)
                rr = g - kk * jnp.int32(length)
                kv = jnp.broadcast_to(kk, (lanes,))
                for ch in range(b // lanes):
                    bbv = lax.iota(jnp.int32, lanes) + jnp.int32(ch * lanes)
                    vals = plsc.load_gather(idxv2, [bbv, kv])
                    idxg[pl.ds(i * b + ch * lanes, lanes)] = (
                        rr * jnp.int32(p) + vals)
            for grp in range(bpw // 4):
                bb0 = wid * bpw + grp * 4
                for ch in range(2):
                    bv = jnp.broadcast_to(bb0 + 2 * ch, (lanes,)) + bqv[...]
                    vals = plsc.load_gather(idxv2, [bv, kcv[...]])
                    idxb[pl.ds(grp * 32 + ch * lanes, lanes)] = vals

            chunks = []
            for i in range(gpw):
                g = wid * gpw + i
                for q in range(nq):
                    chunks.append((
                        lambda i=i, q=q: prompt_hbm.at[
                            idxg.at[pl.ds(i * b + q * qsz, qsz)]],
                        lambda g=g, q=q: out_hbm.at[g, pl.ds(q * qsz, qsz), :],
                    ))
            for grp in range(bpw // 4):
                bb0 = wid * bpw + grp * 4
                chunks.append((
                    lambda grp=grp: pn_hbm.at[
                        idxb.at[pl.ds(grp * 32, 32)]],
                    lambda bb0=bb0: bkn_hbm.at[pl.ds(bb0 * TOPK, 32), :],
                ))

            pend = [None, None]
            for t, (src_fn, dst_fn) in enumerate(chunks):
                buf = bufs[t % 2]
                if pend[t % 2] is not None:
                    pend[t % 2].wait()
                pltpu.async_copy(src_fn(), buf, gsem).wait()
                pend[t % 2] = pltpu.async_copy(buf, dst_fn(), wsem)
            pend[0].wait()
            pend[1].wait()

        pl.run_scoped(run,
                      pltpu.VMEM((qsz, c), jnp.float32),
                      pltpu.VMEM((qsz, c), jnp.float32),
                      pltpu.VMEM((b, TOPK), jnp.int32),
                      pltpu.VMEM((gpw * b,), jnp.int32),
                      pltpu.VMEM((2 * 32,), jnp.int32),
                      pltpu.VMEM((lanes,), jnp.int32),
                      pltpu.VMEM((lanes,), jnp.int32),
                      pltpu.SemaphoreType.DMA,
                      pltpu.SemaphoreType.DMA)

    return pl.kernel(
        body,
        out_type=jax.ShapeDtypeStruct((b * TOPK, c), jnp.float32),
        mesh=mesh,
        compiler_params=pltpu.CompilerParams(needs_layout_passes=False),
    )


def kernel(x_embed, prompt_key, prompt):
    b, n, c = x_embed.shape
    p = prompt_key.shape[0]
    length = prompt.shape[1]
    grows = TOPK * length
    out_rows = grows + n

    xt = jnp.transpose(x_embed, (1, 0, 2))
    prompt_t = jnp.transpose(prompt, (1, 0, 2))
    prompt2d = prompt_t.reshape(length * p, c)

    in_specs = [
        pl.BlockSpec((4, b, c), lambda i: (i, 0, 0)),
        pl.BlockSpec((p, c), lambda i: (0, 0)),
    ]
    out_shapes = (
        jax.ShapeDtypeStruct((out_rows, b, c), jnp.float32),
        jax.ShapeDtypeStruct((b, p), jnp.float32),
        jax.ShapeDtypeStruct((b, TOPK), jnp.int32),
        jax.ShapeDtypeStruct((b, c), jnp.float32),
        jax.ShapeDtypeStruct((p, c), jnp.float32),
        jax.ShapeDtypeStruct((1, 1), jnp.float32),
    )
    out_specs = (
        pl.BlockSpec((4, b, c), lambda i: (grows // 4 + i, 0, 0)),
        pl.BlockSpec((b, p), lambda i: (0, 0)),
        pl.BlockSpec((b, TOPK), lambda i: (0, 0)),
        pl.BlockSpec((b, c), lambda i: (0, 0)),
        pl.BlockSpec((p, c), lambda i: (0, 0)),
        pl.BlockSpec((1, 1), lambda i: (0, 0)),
    )
    outt0, sim, idx, xn, pn, rs = pl.pallas_call(
        functools.partial(_stats_kernel, n=n),
        grid=(n // 4,),
        in_specs=in_specs,
        out_specs=out_specs,
        out_shape=out_shapes,
        scratch_shapes=[pltpu.VMEM((b, c), jnp.float32)],
    )(xt, prompt_key)

    kcm = jnp.arange(16, dtype=jnp.int32) % jnp.int32(TOPK)
    bqm = jnp.arange(16, dtype=jnp.int32) // jnp.int32(TOPK)
    out_ref = jax.new_ref(outt0)
    assemble = _make_assemble(b, n, c, p, length)
    bkn2 = assemble(prompt2d, pn, idx, kcm, bqm, out_ref)
    bkn = bkn2.reshape(b, TOPK, c)
    prompted = jnp.transpose(out_ref[...], (1, 0, 2))

    return (prompted,
            sim,
            rs.reshape(()),
            idx,
            pn,
            xn,
            bkn)


# speedup vs baseline: 1.3569x; 1.0007x over previous
import functools

import jax
import jax.numpy as jnp
from jax import lax
from jax.experimental import pallas as pl
from jax.experimental.pallas import tpu as pltpu
from jax.experimental.pallas import tpu_sc as plsc

TOPK = 8
NUM_WORKERS = 32


def _stats_kernel(xt_ref, pk_ref, out_ref, sim_ref, idx_ref, xn_ref, pn_ref,
                  rs_ref, means_ref, *, n):
    i = pl.program_id(0)
    p = pk_ref.shape[0]
    b = means_ref.shape[0]

    x = xt_ref[...]
    out_ref[...] = x

    @pl.when(i == 0)
    def _init():
        means_ref[...] = jnp.zeros_like(means_ref)

    means_ref[...] += jnp.sum(x, axis=0)

    @pl.when(i == pl.num_programs(0) - 1)
    def _tail():
        pk = pk_ref[...]
        pss = jnp.sum(pk * pk, axis=1, keepdims=True)
        pn = pk * lax.rsqrt(jnp.maximum(pss, jnp.float32(1e-12)))
        pn_ref[...] = pn

        mm = means_ref[...] / jnp.float32(n)
        mss = jnp.sum(mm * mm, axis=1, keepdims=True)
        xn = mm * lax.rsqrt(jnp.maximum(mss, jnp.float32(1e-12)))
        xn_ref[...] = xn

        sim = lax.dot_general(
            xn, pn, (((1,), (1,)), ((), ())),
            precision=lax.Precision.DEFAULT,
            preferred_element_type=jnp.float32)
        sim_ref[...] = sim

        iota = lax.broadcasted_iota(jnp.int32, (b, p), 1)
        kiota = lax.broadcasted_iota(jnp.int32, (b, TOPK), 1)
        vals = sim
        idx_acc = jnp.zeros((b, TOPK), jnp.int32)
        ssum = jnp.float32(0.0)
        for k in range(TOPK):
            m = jnp.max(vals, axis=1, keepdims=True)
            im = jnp.min(jnp.where(vals == m, iota, jnp.int32(p)),
                         axis=1, keepdims=True)
            idx_acc = jnp.where(kiota == k, im, idx_acc)
            ssum = ssum + jnp.sum(m)
            vals = jnp.where(iota == im, -jnp.inf, vals)
        idx_ref[...] = idx_acc
        rs_ref[...] = jnp.full((1, 1), ssum / jnp.float32(b), jnp.float32)


def _make_assemble(b, n, c, p, length):
    grows = TOPK * length
    out_rows = grows + n
    qsz = b // 8
    nq = b // qsz
    lanes = 16
    gpw = grows // NUM_WORKERS
    cpw = -(-n // NUM_WORKERS)
    bpw = b // NUM_WORKERS

    mesh = plsc.VectorSubcoreMesh(core_axis_name="c", subcore_axis_name="s",
                                  num_cores=2, num_subcores=16)

    def body(prompt_hbm, pn_hbm, idx_hbm, kcm_hbm, bqm_hbm, out_hbm,
             bkn_hbm):
        wid = lax.axis_index("s") * 2 + lax.axis_index("c")

        def run(h0, h1, h2, idxv2, idxg, idxb, kcv, bqv, gsem, wsem):
            bufs = (h0, h1, h2)
            nbuf = len(bufs)
            pltpu.sync_copy(kcm_hbm, kcv)
            pltpu.sync_copy(bqm_hbm, bqv)
            pltpu.sync_copy(idx_hbm, idxv2)

            for i in range(gpw):
                g = wid * gpw + i
                kk = g // jnp.int32(length)
                rr = g - kk * jnp.int32(length)
                kv = jnp.broadcast_to(kk, (lanes,))
                for ch in range(b // lanes):
                    bbv = lax.iota(jnp.int32, lanes) + jnp.int32(ch * lanes)
                    vals = plsc.load_gather(idxv2, [bbv, kv])
                    idxg[pl.ds(i * b + ch * lanes, lanes)] = (
                        rr * jnp.int32(p) + vals)
            for grp in range(bpw // 4):
                bb0 = wid * bpw + grp * 4
                for ch in range(2):
                    bv = jnp.broadcast_to(bb0 + 2 * ch, (lanes,)) + bqv[...]
                    vals = plsc.load_gather(idxv2, [bv, kcv[...]])
                    idxb[pl.ds(grp * 32 + ch * lanes, lanes)] = vals

            chunks = []
            for i in range(gpw):
                g = wid * gpw + i
                for q in range(nq):
                    chunks.append((
                        lambda i=i, q=q: prompt_hbm.at[
                            idxg.at[pl.ds(i * b + q * qsz, qsz)]],
                        lambda g=g, q=q: out_hbm.at[g, pl.ds(q * qsz, qsz), :],
                    ))
            for grp in range(bpw // 4):
                bb0 = wid * bpw + grp * 4
                chunks.append((
                    lambda grp=grp: pn_hbm.at[
                        idxb.at[pl.ds(grp * 32, 32)]],
                    lambda bb0=bb0: bkn_hbm.at[pl.ds(bb0 * TOPK, 32), :],
                ))

            pend = [None] * nbuf
            for t, (src_fn, dst_fn) in enumerate(chunks):
                buf = bufs[t % nbuf]
                if pend[t % nbuf] is not None:
                    pend[t % nbuf].wait()
                pltpu.async_copy(src_fn(), buf, gsem).wait()
                pend[t % nbuf] = pltpu.async_copy(buf, dst_fn(), wsem)
            for w in pend:
                if w is not None:
                    w.wait()

        pl.run_scoped(run,
                      pltpu.VMEM((qsz, c), jnp.float32),
                      pltpu.VMEM((qsz, c), jnp.float32),
                      pltpu.VMEM((qsz, c), jnp.float32),
                      pltpu.VMEM((b, TOPK), jnp.int32),
                      pltpu.VMEM((gpw * b,), jnp.int32),
                      pltpu.VMEM((2 * 32,), jnp.int32),
                      pltpu.VMEM((lanes,), jnp.int32),
                      pltpu.VMEM((lanes,), jnp.int32),
                      pltpu.SemaphoreType.DMA,
                      pltpu.SemaphoreType.DMA)

    return pl.kernel(
        body,
        out_type=jax.ShapeDtypeStruct((b * TOPK, c), jnp.float32),
        mesh=mesh,
        compiler_params=pltpu.CompilerParams(needs_layout_passes=False),
    )


def kernel(x_embed, prompt_key, prompt):
    b, n, c = x_embed.shape
    p = prompt_key.shape[0]
    length = prompt.shape[1]
    grows = TOPK * length
    out_rows = grows + n

    xt = jnp.transpose(x_embed, (1, 0, 2))
    prompt_t = jnp.transpose(prompt, (1, 0, 2))
    prompt2d = prompt_t.reshape(length * p, c)

    in_specs = [
        pl.BlockSpec((4, b, c), lambda i: (i, 0, 0)),
        pl.BlockSpec((p, c), lambda i: (0, 0)),
    ]
    out_shapes = (
        jax.ShapeDtypeStruct((out_rows, b, c), jnp.float32),
        jax.ShapeDtypeStruct((b, p), jnp.float32),
        jax.ShapeDtypeStruct((b, TOPK), jnp.int32),
        jax.ShapeDtypeStruct((b, c), jnp.float32),
        jax.ShapeDtypeStruct((p, c), jnp.float32),
        jax.ShapeDtypeStruct((1, 1), jnp.float32),
    )
    out_specs = (
        pl.BlockSpec((4, b, c), lambda i: (grows // 4 + i, 0, 0)),
        pl.BlockSpec((b, p), lambda i: (0, 0)),
        pl.BlockSpec((b, TOPK), lambda i: (0, 0)),
        pl.BlockSpec((b, c), lambda i: (0, 0)),
        pl.BlockSpec((p, c), lambda i: (0, 0)),
        pl.BlockSpec((1, 1), lambda i: (0, 0)),
    )
    outt0, sim, idx, xn, pn, rs = pl.pallas_call(
        functools.partial(_stats_kernel, n=n),
        grid=(n // 4,),
        in_specs=in_specs,
        out_specs=out_specs,
        out_shape=out_shapes,
        scratch_shapes=[pltpu.VMEM((b, c), jnp.float32)],
    )(xt, prompt_key)

    kcm = jnp.arange(16, dtype=jnp.int32) % jnp.int32(TOPK)
    bqm = jnp.arange(16, dtype=jnp.int32) // jnp.int32(TOPK)
    out_ref = jax.new_ref(outt0)
    assemble = _make_assemble(b, n, c, p, length)
    bkn2 = assemble(prompt2d, pn, idx, kcm, bqm, out_ref)
    bkn = bkn2.reshape(b, TOPK, c)
    prompted = jnp.transpose(out_ref[...], (1, 0, 2))

    return (prompted,
            sim,
            rs.reshape(()),
            idx,
            pn,
            xn,
            bkn)


# speedup vs baseline: 1.3587x; 1.0013x over previous
import functools

import jax
import jax.numpy as jnp
from jax import lax
from jax.experimental import pallas as pl
from jax.experimental.pallas import tpu as pltpu
from jax.experimental.pallas import tpu_sc as plsc

TOPK = 8
NUM_WORKERS = 32


def _stats_kernel(xt_ref, pk_ref, out_ref, sim_ref, idx_ref, xn_ref, pn_ref,
                  rs_ref, means_ref, *, n):
    i = pl.program_id(0)
    p = pk_ref.shape[0]
    b = means_ref.shape[0]

    x = xt_ref[...]
    out_ref[...] = x

    @pl.when(i == 0)
    def _init():
        means_ref[...] = jnp.zeros_like(means_ref)

    means_ref[...] += jnp.sum(x, axis=0)

    @pl.when(i == pl.num_programs(0) - 1)
    def _tail():
        pk = pk_ref[...]
        pss = jnp.sum(pk * pk, axis=1, keepdims=True)
        pn = pk * lax.rsqrt(jnp.maximum(pss, jnp.float32(1e-12)))
        pn_ref[...] = pn

        mm = means_ref[...] / jnp.float32(n)
        mss = jnp.sum(mm * mm, axis=1, keepdims=True)
        xn = mm * lax.rsqrt(jnp.maximum(mss, jnp.float32(1e-12)))
        xn_ref[...] = xn

        sim = lax.dot_general(
            xn, pn, (((1,), (1,)), ((), ())),
            precision=lax.Precision.DEFAULT,
            preferred_element_type=jnp.float32)
        sim_ref[...] = sim

        iota = lax.broadcasted_iota(jnp.int32, (b, p), 1)
        kiota = lax.broadcasted_iota(jnp.int32, (b, TOPK), 1)
        vals = sim
        idx_acc = jnp.zeros((b, TOPK), jnp.int32)
        ssum = jnp.float32(0.0)
        for k in range(TOPK):
            m = jnp.max(vals, axis=1, keepdims=True)
            im = jnp.min(jnp.where(vals == m, iota, jnp.int32(p)),
                         axis=1, keepdims=True)
            idx_acc = jnp.where(kiota == k, im, idx_acc)
            ssum = ssum + jnp.sum(m)
            vals = jnp.where(iota == im, -jnp.inf, vals)
        idx_ref[...] = idx_acc
        rs_ref[...] = jnp.full((1, 1), ssum / jnp.float32(b), jnp.float32)


def _make_assemble(b, n, c, p, length):
    grows = TOPK * length
    out_rows = grows + n
    qsz = b // 8
    nq = b // qsz
    lanes = 16
    gpw = grows // NUM_WORKERS
    bpw = b // NUM_WORKERS

    mesh = plsc.VectorSubcoreMesh(core_axis_name="c", subcore_axis_name="s",
                                  num_cores=2, num_subcores=16)

    def body(prompt_hbm, pn_hbm, idx_hbm, kcm_hbm, bqm_hbm, out_hbm,
             bkn_hbm):
        wid = lax.axis_index("s") * 2 + lax.axis_index("c")

        def run(h0, h1, h2, idxv2, idxg, idxb, kcv, bqv, gsem, wsem):
            bufs = (h0, h1, h2)
            nbuf = len(bufs)
            pltpu.sync_copy(kcm_hbm, kcv)
            pltpu.sync_copy(bqm_hbm, bqv)
            pltpu.sync_copy(idx_hbm, idxv2)

            for i in range(gpw):
                g = wid * gpw + i
                kk = g // jnp.int32(length)
                rr = g - kk * jnp.int32(length)
                kv = jnp.broadcast_to(kk, (lanes,))
                for ch in range(b // lanes):
                    bbv = lax.iota(jnp.int32, lanes) + jnp.int32(ch * lanes)
                    vals = plsc.load_gather(idxv2, [bbv, kv])
                    idxg[pl.ds(i * b + ch * lanes, lanes)] = (
                        rr * jnp.int32(p) + vals)
            for grp in range(bpw // 4):
                bb0 = wid * bpw + grp * 4
                for ch in range(2):
                    bv = jnp.broadcast_to(bb0 + 2 * ch, (lanes,)) + bqv[...]
                    vals = plsc.load_gather(idxv2, [bv, kcv[...]])
                    idxb[pl.ds(grp * 32 + ch * lanes, lanes)] = vals

            chunks = []
            for i in range(gpw):
                g = wid * gpw + i
                for q in range(nq):
                    chunks.append((
                        lambda i=i, q=q: prompt_hbm.at[
                            idxg.at[pl.ds(i * b + q * qsz, qsz)]],
                        lambda g=g, q=q: out_hbm.at[g, pl.ds(q * qsz, qsz), :],
                    ))
            for grp in range(bpw // 4):
                bb0 = wid * bpw + grp * 4
                chunks.append((
                    lambda grp=grp: pn_hbm.at[
                        idxb.at[pl.ds(grp * 32, 32)]],
                    lambda bb0=bb0: bkn_hbm.at[pl.ds(bb0 * TOPK, 32), :],
                ))

            pend = [None] * nbuf
            for t, (src_fn, dst_fn) in enumerate(chunks):
                buf = bufs[t % nbuf]
                if pend[t % nbuf] is not None:
                    pend[t % nbuf].wait()
                pltpu.async_copy(src_fn(), buf, gsem).wait()
                pend[t % nbuf] = pltpu.async_copy(buf, dst_fn(), wsem)
            for w in pend:
                if w is not None:
                    w.wait()

        pl.run_scoped(run,
                      pltpu.VMEM((qsz, c), jnp.float32),
                      pltpu.VMEM((qsz, c), jnp.float32),
                      pltpu.VMEM((qsz, c), jnp.float32),
                      pltpu.VMEM((b, TOPK), jnp.int32),
                      pltpu.VMEM((gpw * b,), jnp.int32),
                      pltpu.VMEM((2 * 32,), jnp.int32),
                      pltpu.VMEM((lanes,), jnp.int32),
                      pltpu.VMEM((lanes,), jnp.int32),
                      pltpu.SemaphoreType.DMA,
                      pltpu.SemaphoreType.DMA)

    return pl.kernel(
        body,
        out_type=jax.ShapeDtypeStruct((b * TOPK, c), jnp.float32),
        mesh=mesh,
        compiler_params=pltpu.CompilerParams(needs_layout_passes=False),
    )


def kernel(x_embed, prompt_key, prompt):
    b, n, c = x_embed.shape
    p = prompt_key.shape[0]
    length = prompt.shape[1]
    grows = TOPK * length
    out_rows = grows + n

    xt = jnp.transpose(x_embed, (1, 0, 2))
    prompt_t = jnp.transpose(prompt, (1, 0, 2))
    prompt2d = prompt_t.reshape(length * p, c)

    in_specs = [
        pl.BlockSpec((4, b, c), lambda i: (i, 0, 0)),
        pl.BlockSpec((p, c), lambda i: (0, 0)),
    ]
    out_shapes = (
        jax.ShapeDtypeStruct((out_rows, b, c), jnp.float32),
        jax.ShapeDtypeStruct((b, p), jnp.float32),
        jax.ShapeDtypeStruct((b, TOPK), jnp.int32),
        jax.ShapeDtypeStruct((b, c), jnp.float32),
        jax.ShapeDtypeStruct((p, c), jnp.float32),
        jax.ShapeDtypeStruct((1, 1), jnp.float32),
    )
    out_specs = (
        pl.BlockSpec((4, b, c), lambda i: (grows // 4 + i, 0, 0)),
        pl.BlockSpec((b, p), lambda i: (0, 0)),
        pl.BlockSpec((b, TOPK), lambda i: (0, 0)),
        pl.BlockSpec((b, c), lambda i: (0, 0)),
        pl.BlockSpec((p, c), lambda i: (0, 0)),
        pl.BlockSpec((1, 1), lambda i: (0, 0)),
    )
    outt0, sim, idx, xn, pn, rs = pl.pallas_call(
        functools.partial(_stats_kernel, n=n),
        grid=(n // 4,),
        in_specs=in_specs,
        out_specs=out_specs,
        out_shape=out_shapes,
        scratch_shapes=[pltpu.VMEM((b, c), jnp.float32)],
    )(xt, prompt_key)

    kcm = jnp.arange(16, dtype=jnp.int32) % jnp.int32(TOPK)
    bqm = jnp.arange(16, dtype=jnp.int32) // jnp.int32(TOPK)
    out_ref = jax.new_ref(outt0)
    assemble = _make_assemble(b, n, c, p, length)
    bkn2 = assemble(prompt2d, pn, idx, kcm, bqm, out_ref)
    bkn = bkn2.reshape(b, TOPK, c)
    prompted = jnp.transpose(out_ref[...], (1, 0, 2))

    return (prompted,
            sim,
            rs.reshape(()),
            idx,
            pn,
            xn,
            bkn)
